# Initial kernel scaffold; baseline (speedup 1.0000x reference)
import jax
import jax.numpy as jnp
from jax.experimental import pallas as pl


def kernel(adj_indices, adj_values, user_emb, item_emb):
    raise NotImplementedError("write your pallas kernel here")


# speedup vs baseline: 4.7312x; 4.7312x over previous
import functools

import jax
import jax.numpy as jnp
from jax import lax
from jax.experimental import pallas as pl
from jax.experimental.pallas import tpu as pltpu
from jax.experimental.pallas import tpu_sc as plsc

NUM_USERS = 2000
NUM_ITEMS = 8000
EMBED_DIM = 128
N_LAYERS = 3
N_NODES = NUM_USERS + NUM_ITEMS
N_EDGES = 320000

NC = 2
NS = 16
L = 16
NW = NC * NS

CHUNK = 128
CHUNKS_TOTAL = N_EDGES // CHUNK
CHUNK_Q, CHUNK_R = divmod(CHUNKS_TOTAL, NW)
ROWS_PER_SUB = 624
TAIL_ROWS = N_NODES - ROWS_PER_SUB * NS
ZROWS = 104


def _sc_layer(table, src, dst, w):
  mesh = plsc.VectorSubcoreMesh(core_axis_name="c", subcore_axis_name="s")

  @functools.partial(
      pl.kernel,
      out_type=jax.ShapeDtypeStruct((NC, N_NODES, EMBED_DIM), jnp.float32),
      mesh=mesh,
      scratch_types=[
          pltpu.VMEM((CHUNK,), jnp.int32),
          pltpu.VMEM((CHUNK,), jnp.int32),
          pltpu.VMEM((CHUNK,), jnp.float32),
          pltpu.VMEM((CHUNK, EMBED_DIM), jnp.float32),
          pltpu.VMEM((ZROWS, EMBED_DIM), jnp.float32),
          pltpu.VMEM_SHARED((N_NODES, EMBED_DIM), jnp.float32),
          pltpu.SemaphoreType.DMA,
      ],
  )
  def k(table_h, src_h, dst_h, w_h, out_h, src_v, dst_v, w_v, rows_v, zero_v,
        acc_sh, sem):
    c = lax.axis_index("c")
    s = lax.axis_index("s")
    wid = s * NC + c

    zeros16 = jnp.zeros((L,), jnp.float32)

    def zbody(i, _):
      for d in range(EMBED_DIM // L):
        zero_v[i, pl.ds(d * L, L)] = zeros16
      return 0

    lax.fori_loop(0, ZROWS, zbody, 0)
    for z in range(ROWS_PER_SUB // ZROWS):
      pltpu.sync_copy(zero_v,
                      acc_sh.at[pl.ds(s * ROWS_PER_SUB + z * ZROWS, ZROWS)])

    @pl.when(s == 0)
    def _():
      pltpu.sync_copy(zero_v.at[pl.ds(0, TAIL_ROWS)],
                      acc_sh.at[pl.ds(ROWS_PER_SUB * NS, TAIL_ROWS)])

    plsc.subcore_barrier()

    n_my = CHUNK_Q + jnp.where(wid < CHUNK_R, 1, 0)
    start_chunk = wid * CHUNK_Q + jnp.minimum(wid, CHUNK_R)

    def body(i, _):
      base = (start_chunk + i) * CHUNK
      pltpu.sync_copy(src_h.at[pl.ds(base, CHUNK)], src_v)
      pltpu.sync_copy(dst_h.at[pl.ds(base, CHUNK)], dst_v)
      pltpu.sync_copy(w_h.at[pl.ds(base, CHUNK)], w_v)
      pltpu.async_copy(table_h.at[src_v], rows_v, sem).wait()

      def scale(g, _):
        wg = w_v[pl.ds(g * L, L)]
        for j in range(L):
          e = g * L + j
          wsp = jnp.full((L,), wg[j], jnp.float32)
          for d in range(EMBED_DIM // L):
            rows_v[e, pl.ds(d * L, L)] = rows_v[e, pl.ds(d * L, L)] * wsp
        return 0

      lax.fori_loop(0, CHUNK // L, scale, 0)
      pltpu.sync_copy(rows_v, acc_sh.at[dst_v], add=True)
      return 0

    lax.fori_loop(0, n_my, body, 0)

    plsc.subcore_barrier()
    pltpu.sync_copy(acc_sh.at[pl.ds(s * ROWS_PER_SUB, ROWS_PER_SUB)],
                    out_h.at[c, pl.ds(s * ROWS_PER_SUB, ROWS_PER_SUB)])

    @pl.when(s == 0)
    def _():
      pltpu.sync_copy(acc_sh.at[pl.ds(ROWS_PER_SUB * NS, TAIL_ROWS)],
                      out_h.at[c, pl.ds(ROWS_PER_SUB * NS, TAIL_ROWS)])

  return k(table, src, dst, w)


def _combine(partials, acc, final):
  scale = 0.25 if final else 1.0
  nb = 10
  blk = N_NODES // nb

  def body(p_ref, a_ref, t_ref, o_ref):
    t = p_ref[0] + p_ref[1]
    t_ref[...] = t
    o_ref[...] = (a_ref[...] + t) * scale

  return pl.pallas_call(
      body,
      grid=(nb,),
      in_specs=[
          pl.BlockSpec((2, blk, EMBED_DIM), lambda i: (0, i, 0)),
          pl.BlockSpec((blk, EMBED_DIM), lambda i: (i, 0)),
      ],
      out_specs=[pl.BlockSpec((blk, EMBED_DIM), lambda i: (i, 0))] * 2,
      out_shape=[jax.ShapeDtypeStruct((N_NODES, EMBED_DIM), jnp.float32)] * 2,
  )(partials, acc)


def kernel(adj_indices, adj_values, user_emb, item_emb):
  all_emb = jnp.concatenate([user_emb, item_emb], axis=0)
  dst = adj_indices[0].astype(jnp.int32)
  src = adj_indices[1].astype(jnp.int32)
  w = adj_values

  t = all_emb
  acc = all_emb
  for layer in range(N_LAYERS):
    partials = _sc_layer(t, src, dst, w)
    t, acc = _combine(partials, acc, final=(layer == N_LAYERS - 1))
  return acc[:NUM_USERS], acc[NUM_USERS:]


# speedup vs baseline: 9.3300x; 1.9720x over previous
import functools

import jax
import jax.numpy as jnp
from jax import lax
from jax.experimental import pallas as pl
from jax.experimental.pallas import tpu as pltpu
from jax.experimental.pallas import tpu_sc as plsc

NUM_USERS = 2000
NUM_ITEMS = 8000
EMBED_DIM = 128
N_LAYERS = 3
N_NODES = NUM_USERS + NUM_ITEMS
N_EDGES = 320000

NC = 2
NS = 16
L = 16
NW = NC * NS

CHUNK = 64
CPT = 160
PAD_E = NW * CPT * CHUNK

ROWS_PER_SUB = 624
TAIL_ROWS = N_NODES - ROWS_PER_SUB * NS
ZROWS = 16


def _sc_layer(table, srcp, dstp, wp):
  mesh = plsc.VectorSubcoreMesh(core_axis_name="c", subcore_axis_name="s")

  @functools.partial(
      pl.kernel,
      out_type=jax.ShapeDtypeStruct((NC, N_NODES, EMBED_DIM), jnp.float32),
      mesh=mesh,
      scratch_types=[
          pltpu.VMEM((CPT * CHUNK,), jnp.int32),
          pltpu.VMEM((CPT, CHUNK), jnp.int32),
          pltpu.VMEM((CHUNK,), jnp.float32),
          pltpu.VMEM((CHUNK,), jnp.float32),
          pltpu.VMEM((CHUNK, EMBED_DIM), jnp.float32),
          pltpu.VMEM((CHUNK, EMBED_DIM), jnp.float32),
          pltpu.VMEM((ZROWS, EMBED_DIM), jnp.float32),
          pltpu.VMEM_SHARED((N_NODES, EMBED_DIM), jnp.float32),
          pltpu.SemaphoreType.DMA,
          pltpu.SemaphoreType.DMA,
      ],
  )
  def k(table_h, src_h, dst_h, w_h, out_h, src_all, dst_all, w0, w1, rows0,
        rows1, zero_v, acc_sh, sem0, sem1):
    c = lax.axis_index("c")
    s = lax.axis_index("s")
    wid = s * NC + c
    eb = wid * (CPT * CHUNK)

    def dpre(i, _):
      o = pl.ds(eb + i * CHUNK, CHUNK)
      v = pl.ds(i * CHUNK, CHUNK)
      pltpu.async_copy(src_h.at[o], src_all.at[v], sem1)
      pltpu.async_copy(dst_h.at[o], dst_all.at[i], sem1)
      return 0

    lax.fori_loop(0, CPT, dpre, 0)

    zeros16 = jnp.zeros((L,), jnp.float32)

    def zbody(i, _):
      for d in range(EMBED_DIM // L):
        zero_v[i, pl.ds(d * L, L)] = zeros16
      return 0

    lax.fori_loop(0, ZROWS, zbody, 0)
    for z in range(ROWS_PER_SUB // ZROWS):
      pltpu.sync_copy(zero_v,
                      acc_sh.at[pl.ds(s * ROWS_PER_SUB + z * ZROWS, ZROWS)])

    @pl.when(s == 0)
    def _():
      pltpu.sync_copy(zero_v.at[pl.ds(0, TAIL_ROWS)],
                      acc_sh.at[pl.ds(ROWS_PER_SUB * NS, TAIL_ROWS)])

    def ddrain(i, _):
      o = pl.ds(eb + i * CHUNK, CHUNK)
      v = pl.ds(i * CHUNK, CHUNK)
      pltpu.make_async_copy(src_h.at[o], src_all.at[v], sem1).wait()
      pltpu.make_async_copy(dst_h.at[o], dst_all.at[i], sem1).wait()
      return 0

    lax.fori_loop(0, CPT, ddrain, 0)
    plsc.subcore_barrier()

    rows = (rows0, rows1)
    wbufs = (w0, w1)
    sems = (sem0, sem1)

    def gather_start(ci, b):
      pltpu.async_copy(w_h.at[pl.ds(eb + ci * CHUNK, CHUNK)], wbufs[b],
                       sems[b])
      pltpu.async_copy(table_h.at[src_all.at[pl.ds(ci * CHUNK, CHUNK)]],
                       rows[b], sems[b])

    def gather_wait(ci, b):
      pltpu.make_async_copy(w_h.at[pl.ds(eb + ci * CHUNK, CHUNK)], wbufs[b],
                            sems[b]).wait()
      pltpu.make_async_copy(table_h.at[src_all.at[pl.ds(ci * CHUNK, CHUNK)]],
                            rows[b], sems[b]).wait()

    def scale_scatter(ci, b):
      rv = rows[b]
      wv = wbufs[b]

      def scale(g, _):
        wg = wv[pl.ds(g * L, L)]
        for j in range(L):
          e = g * L + j
          wsp = jnp.full((L,), wg[j], jnp.float32)
          for d in range(EMBED_DIM // L):
            rv[e, pl.ds(d * L, L)] = rv[e, pl.ds(d * L, L)] * wsp
        return 0

      lax.fori_loop(0, CHUNK // L, scale, 0)
      pltpu.sync_copy(rv, acc_sh.at[dst_all.at[ci]], add=True)

    gather_start(0, 0)

    def pair(p, _):
      i0 = 2 * p
      gather_start(i0 + 1, 1)
      gather_wait(i0, 0)
      scale_scatter(i0, 0)
      nxt = jnp.minimum(i0 + 2, CPT - 1)
      gather_start(nxt, 0)
      gather_wait(i0 + 1, 1)
      scale_scatter(i0 + 1, 1)
      return 0

    lax.fori_loop(0, CPT // 2, pair, 0)
    gather_wait(CPT - 1, 0)

    plsc.subcore_barrier()
    pltpu.sync_copy(acc_sh.at[pl.ds(s * ROWS_PER_SUB, ROWS_PER_SUB)],
                    out_h.at[c, pl.ds(s * ROWS_PER_SUB, ROWS_PER_SUB)])

    @pl.when(s == 0)
    def _():
      pltpu.sync_copy(acc_sh.at[pl.ds(ROWS_PER_SUB * NS, TAIL_ROWS)],
                      out_h.at[c, pl.ds(ROWS_PER_SUB * NS, TAIL_ROWS)])

  return k(table, srcp, dstp, wp)


def _combine(partials, acc, final):
  scale = 0.25 if final else 1.0
  nb = 10
  blk = N_NODES // nb

  def body(p_ref, a_ref, t_ref, o_ref):
    t = p_ref[0] + p_ref[1]
    t_ref[...] = t
    o_ref[...] = (a_ref[...] + t) * scale

  return pl.pallas_call(
      body,
      grid=(nb,),
      in_specs=[
          pl.BlockSpec((2, blk, EMBED_DIM), lambda i: (0, i, 0)),
          pl.BlockSpec((blk, EMBED_DIM), lambda i: (i, 0)),
      ],
      out_specs=[pl.BlockSpec((blk, EMBED_DIM), lambda i: (i, 0))] * 2,
      out_shape=[jax.ShapeDtypeStruct((N_NODES, EMBED_DIM), jnp.float32)] * 2,
  )(partials, acc)


def kernel(adj_indices, adj_values, user_emb, item_emb):
  all_emb = jnp.concatenate([user_emb, item_emb], axis=0)
  dst = adj_indices[0].astype(jnp.int32)
  src = adj_indices[1].astype(jnp.int32)

  pad = PAD_E - N_EDGES
  fill = jnp.arange(pad, dtype=jnp.int32) % N_NODES
  srcp = jnp.concatenate([src, fill])
  dstp = jnp.concatenate([dst, fill])
  wp = jnp.concatenate([adj_values, jnp.zeros((pad,), jnp.float32)])

  t = all_emb
  acc = all_emb
  for layer in range(N_LAYERS):
    partials = _sc_layer(t, srcp, dstp, wp)
    t, acc = _combine(partials, acc, final=(layer == N_LAYERS - 1))
  return acc[:NUM_USERS], acc[NUM_USERS:]
